# Initial kernel scaffold; baseline (speedup 1.0000x reference)
import jax
import jax.numpy as jnp
from jax.experimental import pallas as pl


def kernel(x_low, x_high, z_std, params, edge_index_low, edge_index_l2h, edge_index_high):
    raise NotImplementedError("write your pallas kernel here")


# speedup vs baseline: 30.2259x; 30.2259x over previous
import functools

import jax
import jax.numpy as jnp
import numpy as np
from jax import lax
from jax.experimental import pallas as pl
from jax.experimental.pallas import tpu as pltpu
from jax.experimental.pallas import tpu_sc as plsc

N_LOW = 10000
N_HIGH = 50000

N_LOW_PAD = 10240
N_HIGH_PAD = 50176

C_N = 128
B_E = 128


def _bn_affine(p):
    s = p['gamma'] / jnp.sqrt(p['var'] + 1e-5)
    t = p['beta'] - p['mean'] * s
    return s, t


def _cnn_matrices(cnn):
    mats = []
    for i in range(3):
        w = cnn['conv%d' % i]['w']
        bconv = cnn['conv%d' % i]['b']
        s, _ = _bn_affine(cnn['bn%d' % i])
        rows, cols, wnp_idx = [], [], []
        for c in range(5):
            for oi in range(4):
                for oj in range(4):
                    q = c * 16 + oi * 4 + oj
                    for di in range(3):
                        for dj in range(3):
                            ii, jj = oi + di - 1, oj + dj - 1
                            if 0 <= ii < 4 and 0 <= jj < 4:
                                rows.append(c * 16 + ii * 4 + jj)
                                cols.append(q)
                                wnp_idx.append((c, di, dj))
        widx = tuple(np.array(t_) for t_ in zip(*wnp_idx))
        vals = w[:, 0][widx[0], widx[1], widx[2]]
        M = jnp.zeros((80, 80), jnp.float32).at[np.array(rows), np.array(cols)].set(vals)
        scol = jnp.repeat(s, 16)
        bias = jnp.repeat((bconv - cnn['bn%d' % i]['mean']) * s + cnn['bn%d' % i]['beta'], 16)
        mats.append((M * scol[None, :], bias))
    sels = []
    for di in range(2):
        for dj in range(2):
            S = np.zeros((80, 48), np.float32)
            for c in range(5):
                for oi in range(3):
                    for oj in range(3):
                        ii, jj = 2 * oi - 1 + di, 2 * oj - 1 + dj
                        if 0 <= ii < 4 and 0 <= jj < 4:
                            S[c * 16 + ii * 4 + jj, c * 9 + oi * 3 + oj] = 1.0
            sels.append(jnp.asarray(S))
    return mats, sels


def _pad_gat(p, d_in_pad, hc_pad, heads, out_ch, scale=None, shift=None):
    d_src, hc = p['Wl'].shape
    d_dst = p['Wr'].shape[0]
    Wl = jnp.zeros((d_in_pad, hc_pad), jnp.float32).at[:d_src, :hc].set(p['Wl'])
    Wr = jnp.zeros((d_in_pad, hc_pad), jnp.float32).at[:d_dst, :hc].set(p['Wr'])
    bl = jnp.zeros((hc_pad,), jnp.float32).at[:hc].set(p['bl'])
    br = jnp.zeros((hc_pad,), jnp.float32).at[:hc].set(p['br'])
    c_pad = hc_pad // heads
    att = jnp.zeros((heads, c_pad), jnp.float32).at[:, :out_ch].set(p['att'])
    if scale is None:
        scale = jnp.ones((hc,), jnp.float32)
    if shift is None:
        shift = jnp.zeros((hc,), jnp.float32)
    sc = jnp.zeros((hc_pad,), jnp.float32).at[:hc].set(scale)
    sh = jnp.zeros((hc_pad,), jnp.float32).at[:hc].set(p['b'] * scale + shift)
    vecs = jnp.zeros((4, hc_pad), jnp.float32)
    vecs = vecs.at[0].set(att.reshape(-1)).at[1].set(sc).at[2].set(sh)
    return dict(Wl=Wl, Wr=Wr, bl=bl, br=br, vecs=vecs)


def _prep_graph(src, dst, n_dst_pad):
    order = jnp.argsort(dst)
    src_s = src[order].astype(jnp.int32)
    dst_s = dst[order].astype(jnp.int32)
    e = src_s.shape[0]
    num_chunks = n_dst_pad // C_N
    bounds = jnp.searchsorted(
        dst_s, jnp.arange(num_chunks + 1, dtype=jnp.int32) * C_N).astype(jnp.int32)
    cb = jnp.zeros((num_chunks, 16), jnp.int32)
    cb = cb.at[:, 0].set(bounds[:-1]).at[:, 1].set(bounds[1:])
    e_pad = ((e + B_E - 1) // B_E + 1) * B_E + 8
    src_p = jnp.zeros((e_pad,), jnp.int32).at[:e].set(src_s)
    dst_p = jnp.zeros((e_pad,), jnp.int32).at[:e].set(dst_s)
    return src_p, dst_p, cb, num_chunks


def _allsum16(v):
    idx = lax.iota(jnp.int32, 16)
    dnums = lax.GatherDimensionNumbers(
        offset_dims=(), collapsed_slice_dims=(0,), start_index_map=(0,))
    for sh_amt in (8, 4, 2, 1):
        perm = jnp.bitwise_xor(idx, sh_amt)
        v = v + lax.gather(v, perm[:, None], dnums, (1,),
                           mode=lax.GatherScatterMode.PROMISE_IN_BOUNDS)
    return v


@functools.lru_cache(maxsize=None)
def _make_gat_sc(hc, heads, n_src_pad, n_dst_pad, e_pad, num_chunks, relu):
    nv_tot = hc // 16
    c_pad = hc // heads
    nv = c_pad // 16
    nch_per_w = (num_chunks + 31) // 32
    mesh = plsc.VectorSubcoreMesh(core_axis_name="c", subcore_axis_name="s")

    def body(xl, xr, srcs, dsts, cb, vecs, out,
             xr_buf, out_buf, xl_buf, srcb, dstb, cbb, vecs_buf, sem):
        cid = lax.axis_index("c")
        sid = lax.axis_index("s")
        wid = sid * 2 + cid
        pltpu.sync_copy(vecs, vecs_buf)
        att_v = [vecs_buf[0, pl.ds(16 * j, 16)] for j in range(nv_tot)]
        scale_v = [vecs_buf[1, pl.ds(16 * j, 16)] for j in range(nv_tot)]
        shift_v = [vecs_buf[2, pl.ds(16 * j, 16)] for j in range(nv_tot)]
        init_v = [jnp.maximum(sv, 0.0) if relu else sv for sv in shift_v]
        zero16 = jnp.zeros((16,), jnp.float32)
        neg16 = jnp.full((16,), -1e30, jnp.float32)

        def flush(d, cnt, s_t, acc_t):
            cntv = jnp.maximum(jnp.full((16,), cnt, jnp.float32), 1.0)
            for h in range(heads):
                denom = (s_t[h] + 1e-16) * cntv
                for j in range(nv):
                    jj = h * nv + j
                    val = acc_t[jj] / denom * scale_v[jj] + shift_v[jj]
                    if relu:
                        val = jnp.maximum(val, 0.0)
                    out_buf[d, pl.ds(16 * jj, 16)] = val

        def do_chunk(chunk):
            d0 = chunk * C_N
            pltpu.sync_copy(cb.at[chunk], cbb)
            cbv = cbb[...]
            e_lo = cbv[0]
            e_hi = cbv[1]
            pltpu.sync_copy(xr.at[pl.ds(d0, C_N)], xr_buf)

            def initb(d, c):
                for j in range(nv_tot):
                    out_buf[d, pl.ds(16 * j, 16)] = init_v[j]
                return c
            lax.fori_loop(0, C_N, initb, 0)

            e_al = e_lo & (-8)
            nblk = (e_hi - e_al + (B_E - 1)) // B_E

            carry0 = (jnp.int32(-1), jnp.float32(0.0),
                      tuple(neg16 for _ in range(heads)),
                      tuple(zero16 for _ in range(heads)),
                      tuple(zero16 for _ in range(heads * nv)))

            def blk_body(k, carry):
                bstart = pl.multiple_of(e_al + k * B_E, 8)
                pltpu.sync_copy(srcs.at[pl.ds(bstart, B_E)], srcb)
                pltpu.sync_copy(dsts.at[pl.ds(bstart, B_E)],
                                dstb.at[pl.ds(0, B_E)])
                pltpu.async_copy(xl.at[srcb], xl_buf, sem).wait()
                lo_i = jnp.maximum(e_lo - bstart, 0)
                hi_i = jnp.minimum(e_hi - bstart, B_E)

                def edge(i, c2):
                    d_cur, cnt, m_t, s_t, acc_t = c2
                    dl = dstb[pl.ds(i, 16)][0] - d0
                    changed = dl != d_cur

                    @pl.when(jnp.logical_and(changed, d_cur >= 0))
                    def _():
                        flush(d_cur, cnt, s_t, acc_t)

                    keep = jnp.where(changed, 0.0, 1.0)
                    kv = jnp.full((16,), keep, jnp.float32)
                    cnt = cnt * keep
                    m_t = tuple(mv * kv + neg16 * (1.0 - kv) for mv in m_t)
                    s_t = tuple(sv * kv for sv in s_t)
                    acc_t = tuple(av * kv for av in acc_t)

                    new_m, new_s, new_acc = [], [], []
                    for h in range(heads):
                        lacc = zero16
                        xlv = []
                        for j in range(nv):
                            jj = h * nv + j
                            v = xl_buf[i, pl.ds(16 * jj, 16)]
                            r = xr_buf[dl, pl.ds(16 * jj, 16)]
                            ssum = v + r
                            g = jnp.maximum(ssum, 0.0) + 0.2 * jnp.minimum(ssum, 0.0)
                            lacc = lacc + g * att_v[jj]
                            xlv.append(v)
                        av = _allsum16(lacc)
                        mnew = jnp.maximum(m_t[h], av)
                        f = jnp.exp(m_t[h] - mnew)
                        w = jnp.exp(av - mnew)
                        new_m.append(mnew)
                        new_s.append(s_t[h] * f + w)
                        for j in range(nv):
                            jj = h * nv + j
                            new_acc.append(acc_t[jj] * f + xlv[j] * w)
                    return (dl, cnt + 1.0, tuple(new_m), tuple(new_s),
                            tuple(new_acc))

                return lax.fori_loop(lo_i, hi_i, edge, carry)

            d_cur, cnt, m_t, s_t, acc_t = lax.fori_loop(0, nblk, blk_body, carry0)

            @pl.when(d_cur >= 0)
            def _():
                flush(d_cur, cnt, s_t, acc_t)

            pltpu.sync_copy(out_buf, out.at[pl.ds(d0, C_N)])

        def chunk_loop(ci, c):
            chunk = wid + ci * 32

            @pl.when(chunk < num_chunks)
            def _():
                do_chunk(chunk)
            return c
        lax.fori_loop(0, nch_per_w, chunk_loop, 0)

    return pl.kernel(
        body, mesh=mesh,
        out_type=jax.ShapeDtypeStruct((n_dst_pad, hc), jnp.float32),
        scratch_types=[
            pltpu.VMEM((C_N, hc), jnp.float32),
            pltpu.VMEM((C_N, hc), jnp.float32),
            pltpu.VMEM((B_E, hc), jnp.float32),
            pltpu.VMEM((B_E,), jnp.int32),
            pltpu.VMEM((B_E + 16,), jnp.int32),
            pltpu.VMEM((16,), jnp.int32),
            pltpu.VMEM((4, hc), jnp.float32),
            pltpu.SemaphoreType.DMA,
        ],
    )


def _gat_edges_sc(xl, xr, src_s, dst_s, cb, num_chunks, vecs, relu, heads,
                  n_dst_pad):
    hc = xl.shape[1]
    kfn = _make_gat_sc(hc, heads, xl.shape[0], n_dst_pad, src_s.shape[0],
                       num_chunks, relu)
    return kfn(xl, xr, src_s, dst_s, cb, vecs)


_BM = 256


def _tc_proj2(x, Wl, bl, Wr, br):
    M, K = x.shape
    N = Wl.shape[1]

    def body(x_ref, wl_ref, blr, wr_ref, brr, o1, o2):
        xv = x_ref[...]
        o1[...] = jnp.dot(xv, wl_ref[...], preferred_element_type=jnp.float32) + blr[...]
        o2[...] = jnp.dot(xv, wr_ref[...], preferred_element_type=jnp.float32) + brr[...]

    return pl.pallas_call(
        body,
        grid=(M // _BM,),
        in_specs=[pl.BlockSpec((_BM, K), lambda i: (i, 0)),
                  pl.BlockSpec((K, N), lambda i: (0, 0)),
                  pl.BlockSpec((1, N), lambda i: (0, 0)),
                  pl.BlockSpec((K, N), lambda i: (0, 0)),
                  pl.BlockSpec((1, N), lambda i: (0, 0))],
        out_specs=[pl.BlockSpec((_BM, N), lambda i: (i, 0)),
                   pl.BlockSpec((_BM, N), lambda i: (i, 0))],
        out_shape=[jax.ShapeDtypeStruct((M, N), jnp.float32),
                   jax.ShapeDtypeStruct((M, N), jnp.float32)],
    )(x, Wl, bl.reshape(1, -1), Wr, br.reshape(1, -1))


def _tc_mm(x, W, b):
    M, K = x.shape
    N = W.shape[1]

    def body(x_ref, w_ref, b_ref, o_ref):
        o_ref[...] = jnp.dot(x_ref[...], w_ref[...],
                             preferred_element_type=jnp.float32) + b_ref[...]

    return pl.pallas_call(
        body,
        grid=(M // _BM,),
        in_specs=[pl.BlockSpec((_BM, K), lambda i: (i, 0)),
                  pl.BlockSpec((K, N), lambda i: (0, 0)),
                  pl.BlockSpec((1, N), lambda i: (0, 0))],
        out_specs=pl.BlockSpec((_BM, N), lambda i: (i, 0)),
        out_shape=jax.ShapeDtypeStruct((M, N), jnp.float32),
    )(x, W, b.reshape(1, -1))


def _tc_cnn(x80, mats, sels):
    M = x80.shape[0]

    def body(x_ref, m0, b0, m1, b1, m2, b2, s0, s1, s2, s3, o_ref):
        h = x_ref[...]
        for mm, bb in ((m0, b0), (m1, b1), (m2, b2)):
            h = jnp.maximum(jnp.dot(h, mm[...], preferred_element_type=jnp.float32) + bb[...], 0.0)
        c0 = jnp.dot(h, s0[...], preferred_element_type=jnp.float32)
        c1 = jnp.dot(h, s1[...], preferred_element_type=jnp.float32)
        c2 = jnp.dot(h, s2[...], preferred_element_type=jnp.float32)
        c3 = jnp.dot(h, s3[...], preferred_element_type=jnp.float32)
        o_ref[...] = jnp.maximum(jnp.maximum(c0, c1), jnp.maximum(c2, c3))

    args = [x80]
    in_specs = [pl.BlockSpec((_BM, 80), lambda i: (i, 0))]
    for Mw, bw in mats:
        args += [Mw, bw.reshape(1, -1)]
        in_specs += [pl.BlockSpec((80, 80), lambda i: (0, 0)),
                     pl.BlockSpec((1, 80), lambda i: (0, 0))]
    for S in sels:
        args.append(S)
        in_specs.append(pl.BlockSpec((80, 48), lambda i: (0, 0)))

    return pl.pallas_call(
        body,
        grid=(M // _BM,),
        in_specs=in_specs,
        out_specs=pl.BlockSpec((_BM, 48), lambda i: (i, 0)),
        out_shape=jax.ShapeDtypeStruct((M, 48), jnp.float32),
    )(*args)


def _tc_mlp(x, pr):
    M = x.shape[0]
    W1 = jnp.zeros((128, 64), jnp.float32).at[:64].set(pr['W1'])
    W3 = jnp.zeros((32, 8), jnp.float32).at[:, :1].set(pr['W3'])
    b3 = jnp.zeros((8,), jnp.float32).at[:1].set(pr['b3'])

    def body(x_ref, w1, b1, w2, b2, w3, b3r, o_ref):
        y = jnp.maximum(jnp.dot(x_ref[...], w1[...], preferred_element_type=jnp.float32) + b1[...], 0.0)
        y = jnp.maximum(jnp.dot(y, w2[...], preferred_element_type=jnp.float32) + b2[...], 0.0)
        o_ref[...] = jnp.dot(y, w3[...], preferred_element_type=jnp.float32) + b3r[...]

    return pl.pallas_call(
        body,
        grid=(M // _BM,),
        in_specs=[pl.BlockSpec((_BM, 128), lambda i: (i, 0)),
                  pl.BlockSpec((128, 64), lambda i: (0, 0)),
                  pl.BlockSpec((1, 64), lambda i: (0, 0)),
                  pl.BlockSpec((64, 32), lambda i: (0, 0)),
                  pl.BlockSpec((1, 32), lambda i: (0, 0)),
                  pl.BlockSpec((32, 8), lambda i: (0, 0)),
                  pl.BlockSpec((1, 8), lambda i: (0, 0))],
        out_specs=pl.BlockSpec((_BM, 8), lambda i: (i, 0)),
        out_shape=jax.ShapeDtypeStruct((M, 8), jnp.float32),
    )(x, W1, pr['b1'].reshape(1, -1), pr['W2'], pr['b2'].reshape(1, -1),
      W3, b3.reshape(1, -1))


def kernel(x_low, x_high, z_std, params, edge_index_low, edge_index_l2h,
           edge_index_high):
    src_lo, dst_lo, cb_lo, nch_lo = _prep_graph(
        edge_index_low[0], edge_index_low[1], N_LOW_PAD)

    src_l2h, dst_l2h, cb_l2h, nch_hi = _prep_graph(
        edge_index_l2h[0], edge_index_l2h[1], N_HIGH_PAD)

    loop = jnp.arange(N_HIGH, dtype=edge_index_high.dtype)
    sh = jnp.concatenate([edge_index_high[0], loop])
    dh = jnp.concatenate([edge_index_high[1], loop])
    src_hi, dst_hi, cb_hi, _ = _prep_graph(sh, dh, N_HIGH_PAD)

    mats, sels = _cnn_matrices(params['cnn'])
    x80 = jnp.zeros((N_LOW_PAD, 80), jnp.float32).at[:N_LOW].set(
        x_low.reshape(N_LOW, 80))
    h = _tc_cnn(x80, mats, sels)

    for p in params['gl']:
        pp = _pad_gat(p, 128 if p is not params['gl'][0] else 48, 128, 1, 45)
        xl, xr = _tc_proj2(h, pp['Wl'], pp['bl'], pp['Wr'], pp['br'])
        h = _gat_edges_sc(xl, xr, src_lo, dst_lo, cb_lo, nch_lo, pp['vecs'],
                          True, 1, N_LOW_PAD)

    pp = _pad_gat(params['down'], 128, 128, 1, 64)
    xl = _tc_mm(h, pp['Wl'], pp['bl'])
    xh8 = jnp.zeros((N_HIGH_PAD, 8), jnp.float32).at[:N_HIGH, :1].set(x_high)
    Wr8 = jnp.zeros((8, 128), jnp.float32).at[:1, :64].set(params['down']['Wr'])
    xr = _tc_mm(xh8, Wr8, pp['br'])
    h2 = _gat_edges_sc(xl, xr, src_l2h, dst_l2h, cb_l2h, nch_hi, pp['vecs'],
                       False, 1, N_HIGH_PAD)

    z80 = jnp.zeros((N_HIGH_PAD, 80), jnp.float32)
    z80 = z80.at[:N_HIGH, 0:1].set(z_std)
    z80 = z80.at[:, 1:65].set(h2[:, :64])
    s0, t0 = _bn_affine(params['hbn0'])

    hcfg = [(2, 64, 80), (2, 64, 128), (2, 64, 128), (2, 64, 128), (1, 64, 128)]
    x = z80
    for i, (heads, cc, din) in enumerate(hcfg):
        p = params['hg'][i]
        if i < 4:
            sbn, tbn = _bn_affine(params['hbn'][i])
            pp = _pad_gat(p, din, 128, heads, cc, scale=sbn, shift=tbn)
        else:
            pp = _pad_gat(p, din, 128, heads, cc)
        if i == 0:
            sfull = jnp.zeros((80,), jnp.float32).at[:65].set(s0)
            tfull = jnp.zeros((80,), jnp.float32).at[:65].set(t0)
            Wl = pp['Wl'] * sfull[:, None]
            Wr = pp['Wr'] * sfull[:, None]
            bl = pp['bl'] + tfull @ pp['Wl']
            br = pp['br'] + tfull @ pp['Wr']
        else:
            Wl, Wr, bl, br = pp['Wl'], pp['Wr'], pp['bl'], pp['br']
        xl, xr = _tc_proj2(x, Wl, bl, Wr, br)
        x = _gat_edges_sc(xl, xr, src_hi, dst_hi, cb_hi, nch_hi, pp['vecs'],
                          True, heads, N_HIGH_PAD)

    y = _tc_mlp(x, params['pred'])
    return y[:N_HIGH, :1]


# speedup vs baseline: 33.2741x; 1.1008x over previous
import functools

import jax
import jax.numpy as jnp
import numpy as np
from jax import lax
from jax.experimental import pallas as pl
from jax.experimental.pallas import tpu as pltpu
from jax.experimental.pallas import tpu_sc as plsc

N_LOW = 10000
N_HIGH = 50000

N_LOW_PAD = 10240
N_HIGH_PAD = 50176

C_N = 128
B_E = 128


def _bn_affine(p):
    s = p['gamma'] / jnp.sqrt(p['var'] + 1e-5)
    t = p['beta'] - p['mean'] * s
    return s, t


def _cnn_matrices(cnn):
    mats = []
    for i in range(3):
        w = cnn['conv%d' % i]['w']
        bconv = cnn['conv%d' % i]['b']
        s, _ = _bn_affine(cnn['bn%d' % i])
        rows, cols, wnp_idx = [], [], []
        for c in range(5):
            for oi in range(4):
                for oj in range(4):
                    q = c * 16 + oi * 4 + oj
                    for di in range(3):
                        for dj in range(3):
                            ii, jj = oi + di - 1, oj + dj - 1
                            if 0 <= ii < 4 and 0 <= jj < 4:
                                rows.append(c * 16 + ii * 4 + jj)
                                cols.append(q)
                                wnp_idx.append((c, di, dj))
        widx = tuple(np.array(t_) for t_ in zip(*wnp_idx))
        vals = w[:, 0][widx[0], widx[1], widx[2]]
        M = jnp.zeros((80, 80), jnp.float32).at[np.array(rows), np.array(cols)].set(vals)
        scol = jnp.repeat(s, 16)
        bias = jnp.repeat((bconv - cnn['bn%d' % i]['mean']) * s + cnn['bn%d' % i]['beta'], 16)
        mats.append((M * scol[None, :], bias))
    sels = []
    for di in range(2):
        for dj in range(2):
            S = np.zeros((80, 48), np.float32)
            for c in range(5):
                for oi in range(3):
                    for oj in range(3):
                        ii, jj = 2 * oi - 1 + di, 2 * oj - 1 + dj
                        if 0 <= ii < 4 and 0 <= jj < 4:
                            S[c * 16 + ii * 4 + jj, c * 9 + oi * 3 + oj] = 1.0
            sels.append(jnp.asarray(S))
    return mats, sels


def _pad_gat(p, d_in_pad, hc_pad, heads, out_ch, scale=None, shift=None):
    d_src, hc = p['Wl'].shape
    d_dst = p['Wr'].shape[0]
    Wl = jnp.zeros((d_in_pad, hc_pad), jnp.float32).at[:d_src, :hc].set(p['Wl'])
    Wr = jnp.zeros((d_in_pad, hc_pad), jnp.float32).at[:d_dst, :hc].set(p['Wr'])
    bl = jnp.zeros((hc_pad,), jnp.float32).at[:hc].set(p['bl'])
    br = jnp.zeros((hc_pad,), jnp.float32).at[:hc].set(p['br'])
    c_pad = hc_pad // heads
    att = jnp.zeros((heads, c_pad), jnp.float32).at[:, :out_ch].set(p['att'])
    if scale is None:
        scale = jnp.ones((hc,), jnp.float32)
    if shift is None:
        shift = jnp.zeros((hc,), jnp.float32)
    sc = jnp.zeros((hc_pad,), jnp.float32).at[:hc].set(scale)
    sh = jnp.zeros((hc_pad,), jnp.float32).at[:hc].set(p['b'] * scale + shift)
    vecs = jnp.zeros((4, hc_pad), jnp.float32)
    vecs = vecs.at[0].set(att.reshape(-1)).at[1].set(sc).at[2].set(sh)
    return dict(Wl=Wl, Wr=Wr, bl=bl, br=br, vecs=vecs)


def _prep_graph(src, dst, n_dst_pad):
    order = jnp.argsort(dst)
    src_s = src[order].astype(jnp.int32)
    dst_s = dst[order].astype(jnp.int32)
    e = src_s.shape[0]
    num_chunks = n_dst_pad // C_N
    bounds = jnp.searchsorted(
        dst_s, jnp.arange(num_chunks + 1, dtype=jnp.int32) * C_N).astype(jnp.int32)
    cb = jnp.zeros((num_chunks, 16), jnp.int32)
    cb = cb.at[:, 0].set(bounds[:-1]).at[:, 1].set(bounds[1:])
    e_pad = ((e + B_E - 1) // B_E + 4) * B_E + 8
    src_p = jnp.zeros((e_pad,), jnp.int32).at[:e].set(src_s)
    dst_p = jnp.zeros((e_pad,), jnp.int32).at[:e].set(dst_s)
    return src_p, dst_p, cb, num_chunks


def _allsum16(v):
    idx = lax.iota(jnp.int32, 16)
    dnums = lax.GatherDimensionNumbers(
        offset_dims=(), collapsed_slice_dims=(0,), start_index_map=(0,))
    for sh_amt in (8, 4, 2, 1):
        perm = jnp.bitwise_xor(idx, sh_amt)
        v = v + lax.gather(v, perm[:, None], dnums, (1,),
                           mode=lax.GatherScatterMode.PROMISE_IN_BOUNDS)
    return v


@functools.lru_cache(maxsize=None)
def _make_gat_sc(hc, heads, n_src_pad, n_dst_pad, e_pad, num_chunks, relu):
    nv_tot = hc // 16
    c_pad = hc // heads
    nv = c_pad // 16
    nch_per_w = (num_chunks + 31) // 32
    mesh = plsc.VectorSubcoreMesh(core_axis_name="c", subcore_axis_name="s")

    def body(xl, xr, srcs, dsts, cb, vecs, out,
             xr_buf, out_buf, xl_bufA, xl_bufB, srcbA, srcbB, dstbA, dstbB,
             cbb, vecs_buf, semA, semB):
        cid = lax.axis_index("c")
        sid = lax.axis_index("s")
        wid = sid * 2 + cid
        pltpu.sync_copy(vecs, vecs_buf)
        att_v = [vecs_buf[0, pl.ds(16 * j, 16)] for j in range(nv_tot)]
        scale_v = [vecs_buf[1, pl.ds(16 * j, 16)] for j in range(nv_tot)]
        shift_v = [vecs_buf[2, pl.ds(16 * j, 16)] for j in range(nv_tot)]
        init_v = [jnp.maximum(sv, 0.0) if relu else sv for sv in shift_v]
        zero16 = jnp.zeros((16,), jnp.float32)
        neg16 = jnp.full((16,), -1e30, jnp.float32)

        def flush(d, cnt, s_t, acc_t):
            cntv = jnp.maximum(jnp.full((16,), cnt, jnp.float32), 1.0)
            for h in range(heads):
                denom = (s_t[h] + 1e-16) * cntv
                for j in range(nv):
                    jj = h * nv + j
                    val = acc_t[jj] / denom * scale_v[jj] + shift_v[jj]
                    if relu:
                        val = jnp.maximum(val, 0.0)
                    out_buf[d, pl.ds(16 * jj, 16)] = val

        def do_chunk(chunk):
            d0 = chunk * C_N
            pltpu.sync_copy(cb.at[chunk], cbb)
            cbv = cbb[...]
            e_lo = cbv[0]
            e_hi = cbv[1]
            pltpu.sync_copy(xr.at[pl.ds(d0, C_N)], xr_buf)

            def initb(d, c):
                for j in range(nv_tot):
                    out_buf[d, pl.ds(16 * j, 16)] = init_v[j]
                return c
            lax.fori_loop(0, C_N, initb, 0)

            e_al = e_lo & (-8)
            nblk = (e_hi - e_al + (B_E - 1)) // B_E
            npair = jnp.maximum((nblk + 1) // 2, 1)

            carry0 = (jnp.int32(-1), jnp.float32(0.0),
                      tuple(neg16 for _ in range(heads)),
                      tuple(zero16 for _ in range(heads)),
                      tuple(zero16 for _ in range(heads * nv)))

            def start_blk(k, srcb_, dstb_, xlb, sem_):
                bstart = pl.multiple_of(e_al + k * B_E, 8)
                pltpu.sync_copy(srcs.at[pl.ds(bstart, B_E)], srcb_)
                pltpu.sync_copy(dsts.at[pl.ds(bstart, B_E)],
                                dstb_.at[pl.ds(0, B_E)])
                pltpu.async_copy(xl.at[srcb_], xlb, sem_)

            def wait_blk(srcb_, xlb, sem_):
                pltpu.make_async_copy(xl.at[srcb_], xlb, sem_).wait()

            def proc(k, dstb_, xlb, carry):
                bstart = e_al + k * B_E
                lo_i = jnp.maximum(e_lo - bstart, 0)
                hi_i = jnp.minimum(e_hi - bstart, B_E)

                def edge(i, c2):
                    d_cur, cnt, m_t, s_t, acc_t = c2
                    dl = dstb_[pl.ds(i, 16)][0] - d0
                    changed = dl != d_cur

                    @pl.when(jnp.logical_and(changed, d_cur >= 0))
                    def _():
                        flush(d_cur, cnt, s_t, acc_t)

                    keep = jnp.where(changed, 0.0, 1.0)
                    kv = jnp.full((16,), keep, jnp.float32)
                    cnt = cnt * keep
                    m_t = tuple(mv * kv + neg16 * (1.0 - kv) for mv in m_t)
                    s_t = tuple(sv * kv for sv in s_t)
                    acc_t = tuple(av * kv for av in acc_t)

                    new_m, new_s, new_acc = [], [], []
                    for h in range(heads):
                        lacc = zero16
                        xlv = []
                        for j in range(nv):
                            jj = h * nv + j
                            v = xlb[i, pl.ds(16 * jj, 16)]
                            r = xr_buf[dl, pl.ds(16 * jj, 16)]
                            ssum = v + r
                            g = jnp.maximum(ssum, 0.0) + 0.2 * jnp.minimum(ssum, 0.0)
                            lacc = lacc + g * att_v[jj]
                            xlv.append(v)
                        av = _allsum16(lacc)
                        mnew = jnp.maximum(m_t[h], av)
                        f = jnp.exp(m_t[h] - mnew)
                        w = jnp.exp(av - mnew)
                        new_m.append(mnew)
                        new_s.append(s_t[h] * f + w)
                        for j in range(nv):
                            jj = h * nv + j
                            new_acc.append(acc_t[jj] * f + xlv[j] * w)
                    return (dl, cnt + 1.0, tuple(new_m), tuple(new_s),
                            tuple(new_acc))

                return lax.fori_loop(lo_i, hi_i, edge, carry)

            start_blk(0, srcbA, dstbA, xl_bufA, semA)

            def pair_body(g, carry):
                kA = 2 * g
                start_blk(kA + 1, srcbB, dstbB, xl_bufB, semB)
                wait_blk(srcbA, xl_bufA, semA)
                carry = proc(kA, dstbA, xl_bufA, carry)
                start_blk(kA + 2, srcbA, dstbA, xl_bufA, semA)
                wait_blk(srcbB, xl_bufB, semB)
                carry = proc(kA + 1, dstbB, xl_bufB, carry)
                return carry

            d_cur, cnt, m_t, s_t, acc_t = lax.fori_loop(0, npair, pair_body,
                                                        carry0)
            wait_blk(srcbA, xl_bufA, semA)

            @pl.when(d_cur >= 0)
            def _():
                flush(d_cur, cnt, s_t, acc_t)

            pltpu.sync_copy(out_buf, out.at[pl.ds(d0, C_N)])

        def chunk_loop(ci, c):
            chunk = wid + ci * 32

            @pl.when(chunk < num_chunks)
            def _():
                do_chunk(chunk)
            return c
        lax.fori_loop(0, nch_per_w, chunk_loop, 0)

    return pl.kernel(
        body, mesh=mesh,
        out_type=jax.ShapeDtypeStruct((n_dst_pad, hc), jnp.float32),
        scratch_types=[
            pltpu.VMEM((C_N, hc), jnp.float32),
            pltpu.VMEM((C_N, hc), jnp.float32),
            pltpu.VMEM((B_E, hc), jnp.float32),
            pltpu.VMEM((B_E, hc), jnp.float32),
            pltpu.VMEM((B_E,), jnp.int32),
            pltpu.VMEM((B_E,), jnp.int32),
            pltpu.VMEM((B_E + 16,), jnp.int32),
            pltpu.VMEM((B_E + 16,), jnp.int32),
            pltpu.VMEM((16,), jnp.int32),
            pltpu.VMEM((4, hc), jnp.float32),
            pltpu.SemaphoreType.DMA,
            pltpu.SemaphoreType.DMA,
        ],
    )


def _gat_edges_sc(xl, xr, src_s, dst_s, cb, num_chunks, vecs, relu, heads,
                  n_dst_pad):
    hc = xl.shape[1]
    kfn = _make_gat_sc(hc, heads, xl.shape[0], n_dst_pad, src_s.shape[0],
                       num_chunks, relu)
    return kfn(xl, xr, src_s, dst_s, cb, vecs)


_BM = 256


def _tc_proj2(x, Wl, bl, Wr, br):
    M, K = x.shape
    N = Wl.shape[1]

    def body(x_ref, wl_ref, blr, wr_ref, brr, o1, o2):
        xv = x_ref[...]
        o1[...] = jnp.dot(xv, wl_ref[...], preferred_element_type=jnp.float32) + blr[...]
        o2[...] = jnp.dot(xv, wr_ref[...], preferred_element_type=jnp.float32) + brr[...]

    return pl.pallas_call(
        body,
        grid=(M // _BM,),
        in_specs=[pl.BlockSpec((_BM, K), lambda i: (i, 0)),
                  pl.BlockSpec((K, N), lambda i: (0, 0)),
                  pl.BlockSpec((1, N), lambda i: (0, 0)),
                  pl.BlockSpec((K, N), lambda i: (0, 0)),
                  pl.BlockSpec((1, N), lambda i: (0, 0))],
        out_specs=[pl.BlockSpec((_BM, N), lambda i: (i, 0)),
                   pl.BlockSpec((_BM, N), lambda i: (i, 0))],
        out_shape=[jax.ShapeDtypeStruct((M, N), jnp.float32),
                   jax.ShapeDtypeStruct((M, N), jnp.float32)],
    )(x, Wl, bl.reshape(1, -1), Wr, br.reshape(1, -1))


def _tc_mm(x, W, b):
    M, K = x.shape
    N = W.shape[1]

    def body(x_ref, w_ref, b_ref, o_ref):
        o_ref[...] = jnp.dot(x_ref[...], w_ref[...],
                             preferred_element_type=jnp.float32) + b_ref[...]

    return pl.pallas_call(
        body,
        grid=(M // _BM,),
        in_specs=[pl.BlockSpec((_BM, K), lambda i: (i, 0)),
                  pl.BlockSpec((K, N), lambda i: (0, 0)),
                  pl.BlockSpec((1, N), lambda i: (0, 0))],
        out_specs=pl.BlockSpec((_BM, N), lambda i: (i, 0)),
        out_shape=jax.ShapeDtypeStruct((M, N), jnp.float32),
    )(x, W, b.reshape(1, -1))


def _tc_cnn(x80, mats, sels):
    M = x80.shape[0]

    def body(x_ref, m0, b0, m1, b1, m2, b2, s0, s1, s2, s3, o_ref):
        h = x_ref[...]
        for mm, bb in ((m0, b0), (m1, b1), (m2, b2)):
            h = jnp.maximum(jnp.dot(h, mm[...], preferred_element_type=jnp.float32) + bb[...], 0.0)
        c0 = jnp.dot(h, s0[...], preferred_element_type=jnp.float32)
        c1 = jnp.dot(h, s1[...], preferred_element_type=jnp.float32)
        c2 = jnp.dot(h, s2[...], preferred_element_type=jnp.float32)
        c3 = jnp.dot(h, s3[...], preferred_element_type=jnp.float32)
        o_ref[...] = jnp.maximum(jnp.maximum(c0, c1), jnp.maximum(c2, c3))

    args = [x80]
    in_specs = [pl.BlockSpec((_BM, 80), lambda i: (i, 0))]
    for Mw, bw in mats:
        args += [Mw, bw.reshape(1, -1)]
        in_specs += [pl.BlockSpec((80, 80), lambda i: (0, 0)),
                     pl.BlockSpec((1, 80), lambda i: (0, 0))]
    for S in sels:
        args.append(S)
        in_specs.append(pl.BlockSpec((80, 48), lambda i: (0, 0)))

    return pl.pallas_call(
        body,
        grid=(M // _BM,),
        in_specs=in_specs,
        out_specs=pl.BlockSpec((_BM, 48), lambda i: (i, 0)),
        out_shape=jax.ShapeDtypeStruct((M, 48), jnp.float32),
    )(*args)


def _tc_mlp(x, pr):
    M = x.shape[0]
    W1 = jnp.zeros((128, 64), jnp.float32).at[:64].set(pr['W1'])
    W3 = jnp.zeros((32, 8), jnp.float32).at[:, :1].set(pr['W3'])
    b3 = jnp.zeros((8,), jnp.float32).at[:1].set(pr['b3'])

    def body(x_ref, w1, b1, w2, b2, w3, b3r, o_ref):
        y = jnp.maximum(jnp.dot(x_ref[...], w1[...], preferred_element_type=jnp.float32) + b1[...], 0.0)
        y = jnp.maximum(jnp.dot(y, w2[...], preferred_element_type=jnp.float32) + b2[...], 0.0)
        o_ref[...] = jnp.dot(y, w3[...], preferred_element_type=jnp.float32) + b3r[...]

    return pl.pallas_call(
        body,
        grid=(M // _BM,),
        in_specs=[pl.BlockSpec((_BM, 128), lambda i: (i, 0)),
                  pl.BlockSpec((128, 64), lambda i: (0, 0)),
                  pl.BlockSpec((1, 64), lambda i: (0, 0)),
                  pl.BlockSpec((64, 32), lambda i: (0, 0)),
                  pl.BlockSpec((1, 32), lambda i: (0, 0)),
                  pl.BlockSpec((32, 8), lambda i: (0, 0)),
                  pl.BlockSpec((1, 8), lambda i: (0, 0))],
        out_specs=pl.BlockSpec((_BM, 8), lambda i: (i, 0)),
        out_shape=jax.ShapeDtypeStruct((M, 8), jnp.float32),
    )(x, W1, pr['b1'].reshape(1, -1), pr['W2'], pr['b2'].reshape(1, -1),
      W3, b3.reshape(1, -1))


def kernel(x_low, x_high, z_std, params, edge_index_low, edge_index_l2h,
           edge_index_high):
    src_lo, dst_lo, cb_lo, nch_lo = _prep_graph(
        edge_index_low[0], edge_index_low[1], N_LOW_PAD)

    src_l2h, dst_l2h, cb_l2h, nch_hi = _prep_graph(
        edge_index_l2h[0], edge_index_l2h[1], N_HIGH_PAD)

    loop = jnp.arange(N_HIGH, dtype=edge_index_high.dtype)
    sh = jnp.concatenate([edge_index_high[0], loop])
    dh = jnp.concatenate([edge_index_high[1], loop])
    src_hi, dst_hi, cb_hi, _ = _prep_graph(sh, dh, N_HIGH_PAD)

    mats, sels = _cnn_matrices(params['cnn'])
    x80 = jnp.zeros((N_LOW_PAD, 80), jnp.float32).at[:N_LOW].set(
        x_low.reshape(N_LOW, 80))
    h = _tc_cnn(x80, mats, sels)

    for p in params['gl']:
        pp = _pad_gat(p, 128 if p is not params['gl'][0] else 48, 128, 1, 45)
        xl, xr = _tc_proj2(h, pp['Wl'], pp['bl'], pp['Wr'], pp['br'])
        h = _gat_edges_sc(xl, xr, src_lo, dst_lo, cb_lo, nch_lo, pp['vecs'],
                          True, 1, N_LOW_PAD)

    pp = _pad_gat(params['down'], 128, 128, 1, 64)
    xl = _tc_mm(h, pp['Wl'], pp['bl'])
    xh8 = jnp.zeros((N_HIGH_PAD, 8), jnp.float32).at[:N_HIGH, :1].set(x_high)
    Wr8 = jnp.zeros((8, 128), jnp.float32).at[:1, :64].set(params['down']['Wr'])
    xr = _tc_mm(xh8, Wr8, pp['br'])
    h2 = _gat_edges_sc(xl, xr, src_l2h, dst_l2h, cb_l2h, nch_hi, pp['vecs'],
                       False, 1, N_HIGH_PAD)

    z80 = jnp.zeros((N_HIGH_PAD, 80), jnp.float32)
    z80 = z80.at[:N_HIGH, 0:1].set(z_std)
    z80 = z80.at[:, 1:65].set(h2[:, :64])
    s0, t0 = _bn_affine(params['hbn0'])

    hcfg = [(2, 64, 80), (2, 64, 128), (2, 64, 128), (2, 64, 128), (1, 64, 128)]
    x = z80
    for i, (heads, cc, din) in enumerate(hcfg):
        p = params['hg'][i]
        if i < 4:
            sbn, tbn = _bn_affine(params['hbn'][i])
            pp = _pad_gat(p, din, 128, heads, cc, scale=sbn, shift=tbn)
        else:
            pp = _pad_gat(p, din, 128, heads, cc)
        if i == 0:
            sfull = jnp.zeros((80,), jnp.float32).at[:65].set(s0)
            tfull = jnp.zeros((80,), jnp.float32).at[:65].set(t0)
            Wl = pp['Wl'] * sfull[:, None]
            Wr = pp['Wr'] * sfull[:, None]
            bl = pp['bl'] + tfull @ pp['Wl']
            br = pp['br'] + tfull @ pp['Wr']
        else:
            Wl, Wr, bl, br = pp['Wl'], pp['Wr'], pp['bl'], pp['br']
        xl, xr = _tc_proj2(x, Wl, bl, Wr, br)
        x = _gat_edges_sc(xl, xr, src_hi, dst_hi, cb_hi, nch_hi, pp['vecs'],
                          True, heads, N_HIGH_PAD)

    y = _tc_mlp(x, params['pred'])
    return y[:N_HIGH, :1]
